# baseline (speedup 1.0000x reference)
import functools
import jax
import jax.numpy as jnp
from jax import lax
from jax.experimental import pallas as pl
from jax.experimental.pallas import tpu as pltpu
from jax.experimental.pallas import tpu_sc as plsc

N_IN = 163842
D = 256
N_OUT = (N_IN + 6) // 4
K = 7
L = 16
C = 16
NCHUNK = (N_OUT + C - 1) // C
TAIL = N_OUT - (NCHUNK - 1) * C
NBUF = 3


def _make_kernel():
    info = plsc.get_sparse_core_info()
    NC, NS = info.num_cores, info.num_subcores
    NW = NC * NS
    CPW = -(-NCHUNK // NW)
    assert CPW % NBUF == 0
    NCYC = CPW // NBUF

    mesh = plsc.VectorSubcoreMesh(core_axis_name="c", subcore_axis_name="s")

    @functools.partial(
        pl.kernel,
        mesh=mesh,
        out_type=jax.ShapeDtypeStruct((N_OUT, D), jnp.float32),
        scratch_types=[
            pltpu.VMEM((CPW, C * K), jnp.int32),
            pltpu.VMEM((NBUF, C * K, D), jnp.float32),
            pltpu.VMEM((2, NBUF * C, D), jnp.float32),
            pltpu.SemaphoreType.DMA,
            pltpu.SemaphoreType.DMA,
            pltpu.SemaphoreType.DMA,
            pltpu.SemaphoreType.DMA,
            pltpu.SemaphoreType.DMA,
        ],
    )
    def pool_kernel(x_hbm, idx_hbm, out_hbm, idx_v, rows_v, out_v,
                    g0, g1, g2, o0, o1):
        gsem = [g0, g1, g2]
        osem = [o0, o1]
        wid = lax.axis_index("s") * NC + lax.axis_index("c")
        first = wid * CPW

        pltpu.sync_copy(idx_hbm.at[wid], idx_v)

        def fire_gather(b, t):
            pltpu.async_copy(x_hbm.at[idx_v.at[t]], rows_v.at[b], gsem[b])

        def wait_gather(b):
            pltpu.make_async_copy(
                x_hbm.at[idx_v.at[0]], rows_v.at[b], gsem[b]).wait()

        def fire_batch(p, gbase):
            @pl.when(gbase + NBUF <= NCHUNK)
            def _():
                pltpu.async_copy(out_v.at[p],
                                 out_hbm.at[pl.ds(gbase * C, NBUF * C)],
                                 osem[p])

            @pl.when(gbase + NBUF > NCHUNK)
            def _():
                for b in range(NBUF):
                    g = gbase + b

                    @pl.when(g < NCHUNK - 1)
                    def _():
                        pltpu.async_copy(out_v.at[p, pl.ds(b * C, C)],
                                         out_hbm.at[pl.ds(g * C, C)], osem[p])

                    @pl.when(g == NCHUNK - 1)
                    def _():
                        pltpu.async_copy(out_v.at[p, pl.ds(b * C, TAIL)],
                                         out_hbm.at[pl.ds(g * C, TAIL)],
                                         osem[p])

        def drain_batch(p, gbase):
            @pl.when(gbase + NBUF <= NCHUNK)
            def _():
                pltpu.make_async_copy(
                    out_v.at[p], out_hbm.at[pl.ds(0, NBUF * C)],
                    osem[p]).wait()

            @pl.when(gbase + NBUF > NCHUNK)
            def _():
                for b in range(NBUF):
                    g = gbase + b

                    @pl.when(g < NCHUNK - 1)
                    def _():
                        pltpu.make_async_copy(
                            out_v.at[p, pl.ds(b * C, C)],
                            out_hbm.at[pl.ds(0, C)], osem[p]).wait()

                    @pl.when(g == NCHUNK - 1)
                    def _():
                        pltpu.make_async_copy(
                            out_v.at[p, pl.ds(b * C, TAIL)],
                            out_hbm.at[pl.ds(0, TAIL)], osem[p]).wait()

        def compute(p, b, i, carry):
            for s in range(0, D // L, 4):
                vs = [[rows_v[b, i * K + j, pl.ds((s + q) * L, L)]
                       for j in range(K)] for q in range(4)]
                for q in range(4):
                    v = vs[q]
                    acc = ((v[0] + v[1]) + (v[2] + v[3])) + \
                          ((v[4] + v[5]) + v[6])
                    out_v[p, b * C + i, pl.ds((s + q) * L, L)] = \
                        acc * jnp.float32(1.0 / K)
            return carry

        def cycle(cyc, p):
            tbase = cyc * NBUF
            gbase = first + tbase

            @pl.when(cyc >= 2)
            def _():
                drain_batch(p, gbase - 2 * NBUF)

            for b in range(NBUF):
                t = tbase + b
                g = gbase + b

                @pl.when(g < NCHUNK)
                def _():
                    wait_gather(b)
                    lax.fori_loop(0, C, functools.partial(compute, p, b), 0)

                @pl.when((t + NBUF < CPW) & (g + NBUF < NCHUNK))
                def _():
                    fire_gather(b, t + NBUF)

            @pl.when(gbase < NCHUNK)
            def _():
                fire_batch(p, gbase)

        for b in range(NBUF):
            fire_gather(b, b)

        def step(m, carry):
            cycle(2 * m, 0)
            cycle(2 * m + 1, 1)
            return carry

        lax.fori_loop(0, (NCYC - 1) // 2, step, 0)
        cycle(NCYC - 1, 0)

        @pl.when(first + (NCYC - 2) * NBUF < NCHUNK)
        def _():
            drain_batch(1, first + (NCYC - 2) * NBUF)

        @pl.when(first + (NCYC - 1) * NBUF < NCHUNK)
        def _():
            drain_batch(0, first + (NCYC - 1) * NBUF)

    return pool_kernel


_POOL_KERNEL = _make_kernel()


@jax.jit
def kernel(x, neigh_orders):
    info = plsc.get_sparse_core_info()
    nw = info.num_cores * info.num_subcores
    cpw = -(-NCHUNK // nw)
    idx = neigh_orders[: N_OUT * K]
    pad = nw * cpw * C * K - N_OUT * K
    idx = jnp.concatenate([idx, jnp.zeros((pad,), jnp.int32)])
    return _POOL_KERNEL(x, idx.reshape(nw, cpw, C * K))


# speedup vs baseline: 1.0239x; 1.0239x over previous
import functools
import jax
import jax.numpy as jnp
from jax import lax
from jax.experimental import pallas as pl
from jax.experimental.pallas import tpu as pltpu
from jax.experimental.pallas import tpu_sc as plsc

N_IN = 163842
D = 256
N_OUT = (N_IN + 6) // 4
K = 7
L = 16
C = 16
NCHUNK = (N_OUT + C - 1) // C
TAIL = N_OUT - (NCHUNK - 1) * C
NBUF = 3


def _make_kernel():
    info = plsc.get_sparse_core_info()
    NC, NS = info.num_cores, info.num_subcores
    NW = NC * NS
    CPW = -(-NCHUNK // NW)
    assert CPW % NBUF == 0
    n_pad_chunks = NW * CPW

    mesh = plsc.VectorSubcoreMesh(core_axis_name="c", subcore_axis_name="s")

    @functools.partial(
        pl.kernel,
        mesh=mesh,
        out_type=jax.ShapeDtypeStruct((N_OUT, D), jnp.float32),
        scratch_types=[
            pltpu.VMEM((CPW, C * K), jnp.int32),
            pltpu.VMEM((NBUF, C * K, D), jnp.float32),
            pltpu.VMEM((NBUF, C, D), jnp.float32),
            pltpu.SemaphoreType.DMA,
            pltpu.SemaphoreType.DMA,
            pltpu.SemaphoreType.DMA,
            pltpu.SemaphoreType.DMA,
            pltpu.SemaphoreType.DMA,
            pltpu.SemaphoreType.DMA,
        ],
    )
    def pool_kernel(x_hbm, idx_hbm, out_hbm, idx_v, rows_v, out_v,
                    g0, g1, g2, o0, o1, o2):
        gsem = [g0, g1, g2]
        osem = [o0, o1, o2]
        wid = lax.axis_index("s") * NC + lax.axis_index("c")
        first = wid * CPW

        pltpu.sync_copy(idx_hbm.at[wid], idx_v)

        def fire_gather(b, t):
            pltpu.async_copy(x_hbm.at[idx_v.at[t]], rows_v.at[b], gsem[b])

        def wait_gather(b):
            pltpu.make_async_copy(
                x_hbm.at[idx_v.at[0]], rows_v.at[b], gsem[b]).wait()

        def fire_out(b, g):
            base = g * C

            @pl.when(g < NCHUNK - 1)
            def _():
                pltpu.async_copy(out_v.at[b], out_hbm.at[pl.ds(base, C)],
                                 osem[b])

            @pl.when(g == NCHUNK - 1)
            def _():
                pltpu.async_copy(out_v.at[b, pl.ds(0, TAIL)],
                                 out_hbm.at[pl.ds(base, TAIL)], osem[b])

        def drain_out(b, g_prev):
            @pl.when(g_prev < NCHUNK - 1)
            def _():
                pltpu.make_async_copy(
                    out_v.at[b], out_hbm.at[pl.ds(0, C)], osem[b]).wait()

            @pl.when(g_prev == NCHUNK - 1)
            def _():
                pltpu.make_async_copy(
                    out_v.at[b, pl.ds(0, TAIL)],
                    out_hbm.at[pl.ds(0, TAIL)], osem[b]).wait()

        def compute(b, i, _):
            for s in range(0, D // L, 4):
                vs = [[rows_v[b, i * K + j, pl.ds((s + q) * L, L)]
                       for j in range(K)] for q in range(4)]
                for q in range(4):
                    v = vs[q]
                    acc = ((v[0] + v[1]) + (v[2] + v[3])) + \
                          ((v[4] + v[5]) + v[6])
                    out_v[b, i, pl.ds((s + q) * L, L)] = \
                        acc * jnp.float32(1.0 / K)
            return _

        for b in range(NBUF):
            fire_gather(b, b)

        def step(tt, carry):
            for b in range(NBUF):
                t = tt * NBUF + b
                g = first + t

                @pl.when(t - NBUF >= 0)
                def _():
                    drain_out(b, g - NBUF)

                @pl.when(g < NCHUNK)
                def _():
                    wait_gather(b)
                    lax.fori_loop(0, C, functools.partial(compute, b), 0)
                    fire_out(b, g)

                @pl.when((t + NBUF < CPW) & (g + NBUF < NCHUNK))
                def _():
                    fire_gather(b, t + NBUF)
            return carry

        lax.fori_loop(0, CPW // NBUF, step, 0)

        for b in range(NBUF):
            t = CPW + b
            g_prev = first + t - NBUF

            @pl.when(g_prev < NCHUNK)
            def _():
                drain_out(b, g_prev)

    return pool_kernel


_POOL_KERNEL = _make_kernel()


@jax.jit
def kernel(x, neigh_orders):
    info = plsc.get_sparse_core_info()
    nw = info.num_cores * info.num_subcores
    cpw = -(-NCHUNK // nw)
    idx = neigh_orders[: N_OUT * K]
    pad = nw * cpw * C * K - N_OUT * K
    idx = jnp.concatenate([idx, jnp.zeros((pad,), jnp.int32)])
    return _POOL_KERNEL(x, idx.reshape(nw, cpw, C * K))
